# baseline (speedup 1.0000x reference)
import functools

import jax
import jax.numpy as jnp
from jax import lax
from jax.experimental import pallas as pl
from jax.experimental.pallas import tpu as pltpu
from jax.experimental.pallas import tpu_sc as plsc

NC, NS = 2, 16
NW = NC * NS
LN_EPS_ = 1e-3
N_SPLIT = 2


def _sc_gather_chunk(table, ids, k):
    batch, seq = ids.shape
    h = table.shape[1]
    s_chunk = seq // N_SPLIT
    csize = batch * s_chunk
    b_per_w = csize // NW
    w_per_batch = NW // batch
    chunk = min(32, b_per_w)
    n_chunks = b_per_w // chunk

    mesh = plsc.VectorSubcoreMesh(
        core_axis_name="c", subcore_axis_name="s",
        num_cores=NC, num_subcores=NS)

    @functools.partial(
        pl.kernel,
        mesh=mesh,
        out_type=jax.ShapeDtypeStruct((csize, h), jnp.float32),
        scratch_types=[
            pltpu.VMEM((b_per_w,), jnp.int32),
            pltpu.VMEM((chunk, h), jnp.float32),
            pltpu.VMEM((chunk, h), jnp.float32),
            pltpu.SemaphoreType.DMA,
            pltpu.SemaphoreType.DMA,
        ],
    )
    def gather_kernel(table_hbm, ids_hbm, out_hbm, idx_v, rows0, rows1,
                      sem0, sem1):
        wid = lax.axis_index("s") * NC + lax.axis_index("c")
        row = wid // w_per_batch
        col = k * s_chunk + (wid % w_per_batch) * b_per_w
        dst_base = wid * b_per_w
        bufs = (rows0, rows1)
        sems = (sem0, sem1)

        pltpu.sync_copy(ids_hbm.at[row, pl.ds(col, b_per_w)], idx_v)

        def start(c):
            return pltpu.async_copy(
                table_hbm.at[idx_v.at[pl.ds(c * chunk, chunk)]],
                bufs[c % 2], sems[c % 2])

        cp = start(0)
        for c in range(n_chunks):
            cp.wait()
            if c + 1 < n_chunks:
                cp = start(c + 1)
            pltpu.sync_copy(bufs[c % 2],
                            out_hbm.at[pl.ds(dst_base + c * chunk, chunk)])

    return gather_kernel(table, ids)


def _tc_add_ln_chunk(gathered, tt3, type_emb, pos_emb, gamma, beta,
                     batch, seq, k, out_prev):
    csize, h = gathered.shape
    s_chunk = csize // batch
    t_blk = s_chunk
    grid = csize // t_blk
    blk_per_seq = seq // t_blk

    def body(x_ref, tt_ref, te_ref, pos_ref, g_ref, b_ref, *rest):
        o_ref = rest[-1]
        x = x_ref[...]
        t0 = te_ref[0:1, :]
        dt = te_ref[1:2, :] - t0
        ttf = tt_ref[0, 0, :].astype(jnp.float32).reshape(t_blk, 1)
        x = x + pos_ref[...] + t0 + ttf * dt
        m = jnp.mean(x, axis=-1, keepdims=True)
        d = x - m
        v = jnp.mean(d * d, axis=-1, keepdims=True)
        o_ref[...] = d * lax.rsqrt(v + LN_EPS_) * g_ref[...] + b_ref[...]

    def blk(i):
        return i * blk_per_seq + k

    in_specs = [
        pl.BlockSpec((t_blk, h), lambda i: (i, 0)),
        pl.BlockSpec((1, 1, t_blk), lambda i: (blk(i), 0, 0)),
        pl.BlockSpec((2, h), lambda i: (0, 0)),
        pl.BlockSpec((t_blk, h), lambda i: (k, 0)),
        pl.BlockSpec((1, h), lambda i: (0, 0)),
        pl.BlockSpec((1, h), lambda i: (0, 0)),
    ]
    args = [gathered, tt3, type_emb, pos_emb, gamma, beta]
    aliases = {}
    if out_prev is not None:
        in_specs.append(pl.BlockSpec((8, 128), lambda i: (0, 0)))
        args.append(out_prev)
        aliases = {6: 0}

    return pl.pallas_call(
        body,
        grid=(grid,),
        in_specs=in_specs,
        out_specs=pl.BlockSpec((t_blk, h), lambda i: (blk(i), 0)),
        out_shape=jax.ShapeDtypeStruct((batch * seq, h), jnp.float32),
        input_output_aliases=aliases,
    )(*args)


def kernel(input_ids, token_type_ids, word_embeddings, token_type_embeddings,
           position_embeddings, ln_gamma, ln_beta):
    b, s = input_ids.shape
    h = word_embeddings.shape[1]
    s_chunk = s // N_SPLIT
    ids = input_ids.astype(jnp.int32)
    tt3 = token_type_ids.astype(jnp.int32).reshape(b * s // s_chunk, 1, s_chunk)
    if position_embeddings.shape[0] != s:
        pos = lax.dynamic_slice_in_dim(position_embeddings, 0, s, axis=0)
    else:
        pos = position_embeddings
    gamma = ln_gamma.reshape(1, h)
    beta = ln_beta.reshape(1, h)

    out = None
    for k in range(N_SPLIT):
        g_k = _sc_gather_chunk(word_embeddings, ids, k)
        out = _tc_add_ln_chunk(g_k, tt3, token_type_embeddings, pos,
                               gamma, beta, b, s, k, out)
    return out.reshape(b, s, h)


# speedup vs baseline: 1.0207x; 1.0207x over previous
import functools

import jax
import jax.numpy as jnp
from jax import lax
from jax.experimental import pallas as pl
from jax.experimental.pallas import tpu as pltpu
from jax.experimental.pallas import tpu_sc as plsc

NC, NS = 2, 16
NW = NC * NS
LN_EPS_ = 1e-3
N_SPLIT = 2


def _sc_gather_chunk(table, ids, k):
    batch, seq = ids.shape
    h = table.shape[1]
    s_chunk = seq // N_SPLIT
    csize = batch * s_chunk
    b_per_w = csize // NW
    w_per_batch = NW // batch
    chunk = min(64, b_per_w)
    n_chunks = b_per_w // chunk

    mesh = plsc.VectorSubcoreMesh(
        core_axis_name="c", subcore_axis_name="s",
        num_cores=NC, num_subcores=NS)

    @functools.partial(
        pl.kernel,
        mesh=mesh,
        out_type=jax.ShapeDtypeStruct((csize, h), jnp.float32),
        scratch_types=[
            pltpu.VMEM((b_per_w,), jnp.int32),
            pltpu.VMEM((chunk, h), jnp.float32),
            pltpu.VMEM((chunk, h), jnp.float32),
            pltpu.SemaphoreType.DMA,
            pltpu.SemaphoreType.DMA,
        ],
    )
    def gather_kernel(table_hbm, ids_hbm, out_hbm, idx_v, rows0, rows1,
                      sem0, sem1):
        wid = lax.axis_index("s") * NC + lax.axis_index("c")
        row = wid // w_per_batch
        col = k * s_chunk + (wid % w_per_batch) * b_per_w
        dst_base = wid * b_per_w
        bufs = (rows0, rows1)
        sems = (sem0, sem1)

        pltpu.sync_copy(ids_hbm.at[row, pl.ds(col, b_per_w)], idx_v)

        def start(c):
            return pltpu.async_copy(
                table_hbm.at[idx_v.at[pl.ds(c * chunk, chunk)]],
                bufs[c % 2], sems[c % 2])

        cp = start(0)
        for c in range(n_chunks):
            cp.wait()
            if c + 1 < n_chunks:
                cp = start(c + 1)
            pltpu.sync_copy(bufs[c % 2],
                            out_hbm.at[pl.ds(dst_base + c * chunk, chunk)])

    return gather_kernel(table, ids)


def _tc_add_ln_chunk(gathered, tt3, type_emb, pos_emb, gamma, beta,
                     batch, seq, k, out_prev):
    csize, h = gathered.shape
    s_chunk = csize // batch
    t_blk = s_chunk
    grid = csize // t_blk
    blk_per_seq = seq // t_blk

    def body(x_ref, tt_ref, te_ref, pos_ref, g_ref, b_ref, *rest):
        o_ref = rest[-1]
        x = x_ref[...]
        t0 = te_ref[0:1, :]
        dt = te_ref[1:2, :] - t0
        ttf = tt_ref[0, 0, :].astype(jnp.float32).reshape(t_blk, 1)
        x = x + pos_ref[...] + t0 + ttf * dt
        m = jnp.mean(x, axis=-1, keepdims=True)
        d = x - m
        v = jnp.mean(d * d, axis=-1, keepdims=True)
        o_ref[...] = d * lax.rsqrt(v + LN_EPS_) * g_ref[...] + b_ref[...]

    def blk(i):
        return i * blk_per_seq + k

    in_specs = [
        pl.BlockSpec((t_blk, h), lambda i: (i, 0)),
        pl.BlockSpec((1, 1, t_blk), lambda i: (blk(i), 0, 0)),
        pl.BlockSpec((2, h), lambda i: (0, 0)),
        pl.BlockSpec((t_blk, h), lambda i: (k, 0)),
        pl.BlockSpec((1, h), lambda i: (0, 0)),
        pl.BlockSpec((1, h), lambda i: (0, 0)),
    ]
    args = [gathered, tt3, type_emb, pos_emb, gamma, beta]
    aliases = {}
    if out_prev is not None:
        in_specs.append(pl.BlockSpec((8, 128), lambda i: (0, 0)))
        args.append(out_prev)
        aliases = {6: 0}

    return pl.pallas_call(
        body,
        grid=(grid,),
        in_specs=in_specs,
        out_specs=pl.BlockSpec((t_blk, h), lambda i: (blk(i), 0)),
        out_shape=jax.ShapeDtypeStruct((batch * seq, h), jnp.float32),
        input_output_aliases=aliases,
    )(*args)


def kernel(input_ids, token_type_ids, word_embeddings, token_type_embeddings,
           position_embeddings, ln_gamma, ln_beta):
    b, s = input_ids.shape
    h = word_embeddings.shape[1]
    s_chunk = s // N_SPLIT
    ids = input_ids.astype(jnp.int32)
    tt3 = token_type_ids.astype(jnp.int32).reshape(b * s // s_chunk, 1, s_chunk)
    if position_embeddings.shape[0] != s:
        pos = lax.dynamic_slice_in_dim(position_embeddings, 0, s, axis=0)
    else:
        pos = position_embeddings
    gamma = ln_gamma.reshape(1, h)
    beta = ln_beta.reshape(1, h)

    out = None
    for k in range(N_SPLIT):
        g_k = _sc_gather_chunk(word_embeddings, ids, k)
        out = _tc_add_ln_chunk(g_k, tt3, token_type_embeddings, pos,
                               gamma, beta, b, s, k, out)
    return out.reshape(b, s, h)
